# baseline (speedup 1.0000x reference)
import jax
import jax.numpy as jnp
from jax import lax
from jax.experimental import pallas as pl
from jax.experimental.pallas import tpu as pltpu
from jax.experimental.pallas import tpu_sc as plsc

N_NODES = 10000
N_EDGES = 320000
D_FEAT = 128

NUM_CORES = 2
NUM_SUBCORES = 16
LANES = 16
NW = NUM_CORES * NUM_SUBCORES
EDGES_PER_WORKER = N_EDGES // NW
CHUNK = 64
NCHUNKS = -(-EDGES_PER_WORKER // CHUNK)
GROUPS = CHUNK // LANES
IDX_PER_WORKER = 2 * EDGES_PER_WORKER
IDX_BUF = 2 * CHUNK * NCHUNKS
PACKED = D_FEAT // 2
PACK_CHUNK = 8
ROWS_MAIN = 632


def _edge_decoder(z_src, z_dst, ei_src, ei_dst, par, out_mean, out_std,
                  ps, pd, idx_s, idx_d, u0, v0, u1, v1, mean_v, std_v, par_v,
                  part_v, pin_v, pout_v, sem0, sem1):
  cid = lax.axis_index("c")
  sid = lax.axis_index("s")
  wid = sid * NUM_CORES + cid
  base = wid * EDGES_PER_WORKER

  nch = jnp.where(sid == NUM_SUBCORES - 1,
                  (N_NODES - (NUM_SUBCORES - 1) * ROWS_MAIN) // PACK_CHUNK,
                  ROWS_MAIN // PACK_CHUNK)

  def pack_table(z, dst):
    def chunk_body(i, carry):
      rbase = sid * ROWS_MAIN + i * PACK_CHUNK
      pltpu.sync_copy(z.at[pl.ds(rbase, PACK_CHUNK)], pin_v)

      def row_body(r, carry2):
        for k in range(D_FEAT // (2 * LANES)):
          a = pin_v[r, pl.ds(k * 2 * LANES, LANES)]
          b = pin_v[r, pl.ds(k * 2 * LANES + LANES, LANES)]
          w = plsc.bitcast(
              plsc.pack(a, b, format=plsc.PackFormat.INTERLEAVED), jnp.int32)
          pout_v[r, pl.ds(k * LANES, LANES)] = w
        return carry2

      lax.fori_loop(0, PACK_CHUNK, row_body, 0)
      pltpu.sync_copy(pout_v, dst.at[cid, pl.ds(rbase, PACK_CHUNK)])
      return carry

    lax.fori_loop(0, nch, chunk_body, 0)

  pack_table(z_src, ps)
  pack_table(z_dst, pd)
  plsc.subcore_barrier()

  pltpu.sync_copy(ei_src.at[pl.ds(2 * base, IDX_PER_WORKER)],
                  idx_s.at[pl.ds(0, IDX_PER_WORKER)])
  pltpu.sync_copy(ei_dst.at[pl.ds(2 * base, IDX_PER_WORKER)],
                  idx_d.at[pl.ds(0, IDX_PER_WORKER)])
  zero16 = jnp.zeros((LANES,), jnp.int32)
  for t in range((IDX_BUF - IDX_PER_WORKER) // LANES):
    idx_s[pl.ds(IDX_PER_WORKER + t * LANES, LANES)] = zero16
    idx_d[pl.ds(IDX_PER_WORKER + t * LANES, LANES)] = zero16
  pltpu.sync_copy(par, par_v)
  scale = par_v[0, :]
  bias = par_v[1, :]
  std16 = par_v[2, :]

  def fill_std(j, carry):
    std_v[pl.ds(j * LANES, LANES)] = std16
    return carry

  lax.fori_loop(0, EDGES_PER_WORKER // LANES, fill_std, 0)

  def start_chunk(c, ub, vb, sem):
    pltpu.async_copy(
        ps.at[cid].at[idx_s.at[pl.ds(c * 2 * CHUNK, 2 * CHUNK)]], ub, sem)
    pltpu.async_copy(
        pd.at[cid].at[idx_d.at[pl.ds(c * 2 * CHUNK, 2 * CHUNK)]], vb, sem)

  def wait_chunk(c, ub, vb, sem):
    pltpu.make_async_copy(
        ps.at[cid].at[idx_s.at[pl.ds(c * 2 * CHUNK, 2 * CHUNK)]], ub,
        sem).wait()
    pltpu.make_async_copy(
        pd.at[cid].at[idx_d.at[pl.ds(c * 2 * CHUNK, 2 * CHUNK)]], vb,
        sem).wait()

  def compute_chunk(c, ub, vb):
    stride = lax.iota(jnp.int32, LANES) * LANES

    def group_body(g, carry):
      eb = g * LANES

      def edge_body(e, carry2):
        row = eb + e
        p = []
        for k in range(D_FEAT // (2 * LANES)):
          cu = plsc.bitcast(ub[row, pl.ds(k * LANES, LANES)], jnp.bfloat16)
          cv = plsc.bitcast(vb[row, pl.ds(k * LANES, LANES)], jnp.bfloat16)
          ua, uo = plsc.unpack(cu, format=plsc.PackFormat.INTERLEAVED)
          va, vo = plsc.unpack(cv, format=plsc.PackFormat.INTERLEAVED)
          p.append(ua * va + uo * vo)
        while len(p) > 1:
          p = [p[2 * i] + p[2 * i + 1] for i in range(len(p) // 2)]
        plsc.store_scatter(part_v, [stride + e], p[0])
        return carry2

      lax.fori_loop(0, LANES, edge_body, 0)
      acc = part_v[pl.ds(0, LANES)]
      for l in range(1, LANES):
        acc = acc + part_v[pl.ds(l * LANES, LANES)]
      mean_v[pl.ds(c * CHUNK + eb, LANES)] = acc * scale + bias
      return carry

    lax.fori_loop(0, GROUPS, group_body, 0)

  start_chunk(0, u0, v0, sem0)

  def pair_body(i, carry):
    c0 = 2 * i
    start_chunk(c0 + 1, u1, v1, sem1)
    wait_chunk(c0, u0, v0, sem0)
    compute_chunk(c0, u0, v0)

    @pl.when(c0 + 2 < NCHUNKS)
    def _():
      start_chunk(c0 + 2, u0, v0, sem0)

    wait_chunk(c0 + 1, u1, v1, sem1)
    compute_chunk(c0 + 1, u1, v1)
    return carry

  lax.fori_loop(0, NCHUNKS // 2, pair_body, 0)
  wait_chunk(NCHUNKS - 1, u0, v0, sem0)
  compute_chunk(NCHUNKS - 1, u0, v0)

  pltpu.sync_copy(mean_v.at[pl.ds(0, EDGES_PER_WORKER)],
                  out_mean.at[pl.ds(base, EDGES_PER_WORKER)])
  pltpu.sync_copy(std_v, out_std.at[pl.ds(base, EDGES_PER_WORKER)])


@jax.jit
def _run(z_src, z_dst, ei_src, ei_dst, params):
  mesh = plsc.VectorSubcoreMesh(
      core_axis_name="c", subcore_axis_name="s",
      num_cores=NUM_CORES, num_subcores=NUM_SUBCORES)
  f = pl.kernel(
      _edge_decoder,
      out_type=(jax.ShapeDtypeStruct((N_EDGES,), jnp.float32),
                jax.ShapeDtypeStruct((N_EDGES,), jnp.float32)),
      mesh=mesh,
      compiler_params=pltpu.CompilerParams(needs_layout_passes=False),
      scratch_types=[
          pltpu.HBM((NUM_CORES, N_NODES, PACKED), jnp.int32),
          pltpu.HBM((NUM_CORES, N_NODES, PACKED), jnp.int32),
          pltpu.VMEM((IDX_BUF,), jnp.int32),
          pltpu.VMEM((IDX_BUF,), jnp.int32),
          pltpu.VMEM((2 * CHUNK, PACKED), jnp.int32),
          pltpu.VMEM((2 * CHUNK, PACKED), jnp.int32),
          pltpu.VMEM((2 * CHUNK, PACKED), jnp.int32),
          pltpu.VMEM((2 * CHUNK, PACKED), jnp.int32),
          pltpu.VMEM((CHUNK * NCHUNKS,), jnp.float32),
          pltpu.VMEM((EDGES_PER_WORKER,), jnp.float32),
          pltpu.VMEM((3, LANES), jnp.float32),
          pltpu.VMEM((LANES * LANES,), jnp.float32),
          pltpu.VMEM((PACK_CHUNK, D_FEAT), jnp.float32),
          pltpu.VMEM((PACK_CHUNK, PACKED), jnp.int32),
          pltpu.SemaphoreType.DMA,
          pltpu.SemaphoreType.DMA,
      ],
  )
  mean, std = f(z_src, z_dst, ei_src, ei_dst, params)
  return jnp.stack([mean, std], axis=0)


def kernel(z_src, z_dst, edge_index, src_logscale, src_bias, src_std,
           dst_logscale, dst_bias, dst_std):
  scale = jnp.exp(src_logscale[0] + dst_logscale[0])
  bias = src_bias[0] + dst_bias[0]
  std = jax.nn.softplus(src_std[0]) + jax.nn.softplus(dst_std[0])
  params = jnp.broadcast_to(
      jnp.stack([scale, bias, std])[:, None], (3, LANES))
  def block_ids(ids):
    return jnp.stack([ids * 2, ids * 2 + 1], axis=1).reshape(-1)

  return _run(z_src, z_dst, block_ids(edge_index[0]),
              block_ids(edge_index[1]), params)


# speedup vs baseline: 3.0873x; 3.0873x over previous
import jax
import jax.numpy as jnp
from jax import lax
from jax.experimental import pallas as pl
from jax.experimental.pallas import tpu as pltpu
from jax.experimental.pallas import tpu_sc as plsc

N_NODES = 10000
N_EDGES = 320000
D_FEAT = 128

NUM_CORES = 2
NUM_SUBCORES = 16
LANES = 16
NW = NUM_CORES * NUM_SUBCORES
EDGES_PER_WORKER = N_EDGES // NW
CHUNK = 80
NCHUNKS = EDGES_PER_WORKER // CHUNK
GROUPS = CHUNK // LANES


def _edge_decoder(z_src, z_dst, ei_src, ei_dst, par, out_mean, out_std,
                  idx_s, idx_d, u0, v0, u1, v1, mean_v, std_v, par_v, part_v,
                  sem0, sem1):
  wid = lax.axis_index("s") * NUM_CORES + lax.axis_index("c")
  base = wid * EDGES_PER_WORKER

  pltpu.sync_copy(ei_src.at[pl.ds(base, EDGES_PER_WORKER)], idx_s)
  pltpu.sync_copy(ei_dst.at[pl.ds(base, EDGES_PER_WORKER)], idx_d)
  pltpu.sync_copy(par, par_v)
  scale = par_v[0, :]
  bias = par_v[1, :]
  std16 = par_v[2, :]

  def fill_std(j, carry):
    std_v[pl.ds(j * LANES, LANES)] = std16
    return carry

  lax.fori_loop(0, EDGES_PER_WORKER // LANES, fill_std, 0)

  def start_chunk(c, ub, vb, sem):
    pltpu.async_copy(z_src.at[idx_s.at[pl.ds(c * CHUNK, CHUNK)]], ub, sem)
    pltpu.async_copy(z_dst.at[idx_d.at[pl.ds(c * CHUNK, CHUNK)]], vb, sem)

  def wait_chunk(c, ub, vb, sem):
    pltpu.make_async_copy(
        z_src.at[idx_s.at[pl.ds(c * CHUNK, CHUNK)]], ub, sem).wait()
    pltpu.make_async_copy(
        z_dst.at[idx_d.at[pl.ds(c * CHUNK, CHUNK)]], vb, sem).wait()

  def compute_chunk(c, ub, vb):
    stride = lax.iota(jnp.int32, LANES) * LANES

    def group_body(g, carry):
      eb = g * LANES

      def edge_body(e, carry2):
        row = eb + e
        p = [ub[row, pl.ds(k * LANES, LANES)] * vb[row, pl.ds(k * LANES, LANES)]
             for k in range(D_FEAT // LANES)]
        while len(p) > 1:
          p = [p[2 * i] + p[2 * i + 1] for i in range(len(p) // 2)]
        plsc.store_scatter(part_v, [stride + e], p[0])
        return carry2

      lax.fori_loop(0, LANES, edge_body, 0)
      acc = part_v[pl.ds(0, LANES)]
      for l in range(1, LANES):
        acc = acc + part_v[pl.ds(l * LANES, LANES)]
      mean_v[pl.ds(c * CHUNK + eb, LANES)] = acc * scale + bias
      return carry

    lax.fori_loop(0, GROUPS, group_body, 0)

  start_chunk(0, u0, v0, sem0)

  def pair_body(i, carry):
    c0 = 2 * i
    start_chunk(c0 + 1, u1, v1, sem1)
    wait_chunk(c0, u0, v0, sem0)
    compute_chunk(c0, u0, v0)

    @pl.when(c0 + 2 < NCHUNKS)
    def _():
      start_chunk(c0 + 2, u0, v0, sem0)

    wait_chunk(c0 + 1, u1, v1, sem1)
    compute_chunk(c0 + 1, u1, v1)
    return carry

  lax.fori_loop(0, NCHUNKS // 2, pair_body, 0)
  wait_chunk(NCHUNKS - 1, u0, v0, sem0)
  compute_chunk(NCHUNKS - 1, u0, v0)

  pltpu.sync_copy(mean_v, out_mean.at[pl.ds(base, EDGES_PER_WORKER)])
  pltpu.sync_copy(std_v, out_std.at[pl.ds(base, EDGES_PER_WORKER)])


@jax.jit
def _run(z_src, z_dst, ei_src, ei_dst, params):
  mesh = plsc.VectorSubcoreMesh(
      core_axis_name="c", subcore_axis_name="s",
      num_cores=NUM_CORES, num_subcores=NUM_SUBCORES)
  f = pl.kernel(
      _edge_decoder,
      out_type=(jax.ShapeDtypeStruct((N_EDGES,), jnp.float32),
                jax.ShapeDtypeStruct((N_EDGES,), jnp.float32)),
      mesh=mesh,
      compiler_params=pltpu.CompilerParams(needs_layout_passes=False),
      scratch_types=[
          pltpu.VMEM((EDGES_PER_WORKER,), jnp.int32),
          pltpu.VMEM((EDGES_PER_WORKER,), jnp.int32),
          pltpu.VMEM((CHUNK, D_FEAT), jnp.float32),
          pltpu.VMEM((CHUNK, D_FEAT), jnp.float32),
          pltpu.VMEM((CHUNK, D_FEAT), jnp.float32),
          pltpu.VMEM((CHUNK, D_FEAT), jnp.float32),
          pltpu.VMEM((EDGES_PER_WORKER,), jnp.float32),
          pltpu.VMEM((EDGES_PER_WORKER,), jnp.float32),
          pltpu.VMEM((3, LANES), jnp.float32),
          pltpu.VMEM((LANES * LANES,), jnp.float32),
          pltpu.SemaphoreType.DMA,
          pltpu.SemaphoreType.DMA,
      ],
  )
  mean, std = f(z_src, z_dst, ei_src, ei_dst, params)
  return jnp.stack([mean, std], axis=0)


def kernel(z_src, z_dst, edge_index, src_logscale, src_bias, src_std,
           dst_logscale, dst_bias, dst_std):
  scale = jnp.exp(src_logscale[0] + dst_logscale[0])
  bias = src_bias[0] + dst_bias[0]
  std = jax.nn.softplus(src_std[0]) + jax.nn.softplus(dst_std[0])
  params = jnp.broadcast_to(
      jnp.stack([scale, bias, std])[:, None], (3, LANES))
  return _run(z_src, z_dst, edge_index[0], edge_index[1], params)


# speedup vs baseline: 3.1815x; 1.0305x over previous
import jax
import jax.numpy as jnp
from jax import lax
from jax.experimental import pallas as pl
from jax.experimental.pallas import tpu as pltpu
from jax.experimental.pallas import tpu_sc as plsc

N_NODES = 10000
N_EDGES = 320000
D_FEAT = 128

NUM_CORES = 2
NUM_SUBCORES = 16
LANES = 16
NW = NUM_CORES * NUM_SUBCORES
EDGES_PER_WORKER = N_EDGES // NW
CHUNK = 80
NCHUNKS = EDGES_PER_WORKER // CHUNK
GROUPS = CHUNK // LANES


def _edge_decoder(z_src, z_dst, ei_src, ei_dst, par, out_mean, out_std,
                  idx_s, idx_d, u0, v0, u1, v1, u2, v2, mean_v, std_v, par_v,
                  part_v, sem0, sem1, sem2):
  wid = lax.axis_index("s") * NUM_CORES + lax.axis_index("c")
  base = wid * EDGES_PER_WORKER

  pltpu.sync_copy(ei_src.at[pl.ds(base, EDGES_PER_WORKER)], idx_s)
  pltpu.sync_copy(ei_dst.at[pl.ds(base, EDGES_PER_WORKER)], idx_d)
  pltpu.sync_copy(par, par_v)
  scale = par_v[0, :]
  bias = par_v[1, :]
  std16 = par_v[2, :]

  def fill_std(j, carry):
    std_v[pl.ds(j * LANES, LANES)] = std16
    return carry

  lax.fori_loop(0, EDGES_PER_WORKER // LANES, fill_std, 0)

  def start_chunk(c, ub, vb, sem):
    pltpu.async_copy(z_src.at[idx_s.at[pl.ds(c * CHUNK, CHUNK)]], ub, sem)
    pltpu.async_copy(z_dst.at[idx_d.at[pl.ds(c * CHUNK, CHUNK)]], vb, sem)

  def wait_chunk(c, ub, vb, sem):
    pltpu.make_async_copy(
        z_src.at[idx_s.at[pl.ds(c * CHUNK, CHUNK)]], ub, sem).wait()
    pltpu.make_async_copy(
        z_dst.at[idx_d.at[pl.ds(c * CHUNK, CHUNK)]], vb, sem).wait()

  def compute_chunk(c, ub, vb):
    stride = lax.iota(jnp.int32, LANES) * LANES

    def group_body(g, carry):
      eb = g * LANES

      def edge_body(e, carry2):
        row = eb + e
        p = [ub[row, pl.ds(k * LANES, LANES)] * vb[row, pl.ds(k * LANES, LANES)]
             for k in range(D_FEAT // LANES)]
        while len(p) > 1:
          p = [p[2 * i] + p[2 * i + 1] for i in range(len(p) // 2)]
        plsc.store_scatter(part_v, [stride + e], p[0])
        return carry2

      lax.fori_loop(0, LANES, edge_body, 0)
      acc = part_v[pl.ds(0, LANES)]
      for l in range(1, LANES):
        acc = acc + part_v[pl.ds(l * LANES, LANES)]
      mean_v[pl.ds(c * CHUNK + eb, LANES)] = acc * scale + bias
      return carry

    lax.fori_loop(0, GROUPS, group_body, 0)

  start_chunk(0, u0, v0, sem0)
  start_chunk(1, u1, v1, sem1)

  def triple_body(i, carry):
    c = 3 * i
    start_chunk(c + 2, u2, v2, sem2)
    wait_chunk(c, u0, v0, sem0)
    compute_chunk(c, u0, v0)
    start_chunk(c + 3, u0, v0, sem0)
    wait_chunk(c + 1, u1, v1, sem1)
    compute_chunk(c + 1, u1, v1)
    start_chunk(c + 4, u1, v1, sem1)
    wait_chunk(c + 2, u2, v2, sem2)
    compute_chunk(c + 2, u2, v2)
    return carry

  lax.fori_loop(0, (NCHUNKS - 2) // 3, triple_body, 0)
  wait_chunk(NCHUNKS - 2, u0, v0, sem0)
  compute_chunk(NCHUNKS - 2, u0, v0)
  wait_chunk(NCHUNKS - 1, u1, v1, sem1)
  compute_chunk(NCHUNKS - 1, u1, v1)

  pltpu.sync_copy(mean_v, out_mean.at[pl.ds(base, EDGES_PER_WORKER)])
  pltpu.sync_copy(std_v, out_std.at[pl.ds(base, EDGES_PER_WORKER)])


@jax.jit
def _run(z_src, z_dst, ei_src, ei_dst, params):
  mesh = plsc.VectorSubcoreMesh(
      core_axis_name="c", subcore_axis_name="s",
      num_cores=NUM_CORES, num_subcores=NUM_SUBCORES)
  f = pl.kernel(
      _edge_decoder,
      out_type=(jax.ShapeDtypeStruct((N_EDGES,), jnp.float32),
                jax.ShapeDtypeStruct((N_EDGES,), jnp.float32)),
      mesh=mesh,
      compiler_params=pltpu.CompilerParams(needs_layout_passes=False),
      scratch_types=[
          pltpu.VMEM((EDGES_PER_WORKER,), jnp.int32),
          pltpu.VMEM((EDGES_PER_WORKER,), jnp.int32),
          pltpu.VMEM((CHUNK, D_FEAT), jnp.float32),
          pltpu.VMEM((CHUNK, D_FEAT), jnp.float32),
          pltpu.VMEM((CHUNK, D_FEAT), jnp.float32),
          pltpu.VMEM((CHUNK, D_FEAT), jnp.float32),
          pltpu.VMEM((CHUNK, D_FEAT), jnp.float32),
          pltpu.VMEM((CHUNK, D_FEAT), jnp.float32),
          pltpu.VMEM((EDGES_PER_WORKER,), jnp.float32),
          pltpu.VMEM((EDGES_PER_WORKER,), jnp.float32),
          pltpu.VMEM((3, LANES), jnp.float32),
          pltpu.VMEM((LANES * LANES,), jnp.float32),
          pltpu.SemaphoreType.DMA,
          pltpu.SemaphoreType.DMA,
          pltpu.SemaphoreType.DMA,
      ],
  )
  mean, std = f(z_src, z_dst, ei_src, ei_dst, params)
  return jnp.stack([mean, std], axis=0)


def kernel(z_src, z_dst, edge_index, src_logscale, src_bias, src_std,
           dst_logscale, dst_bias, dst_std):
  scale = jnp.exp(src_logscale[0] + dst_logscale[0])
  bias = src_bias[0] + dst_bias[0]
  std = jax.nn.softplus(src_std[0]) + jax.nn.softplus(dst_std[0])
  params = jnp.broadcast_to(
      jnp.stack([scale, bias, std])[:, None], (3, LANES))
  return _run(z_src, z_dst, edge_index[0], edge_index[1], params)
